# Initial kernel scaffold; baseline (speedup 1.0000x reference)
import jax
import jax.numpy as jnp
from jax.experimental import pallas as pl


def kernel(dense, cats, tables, W_lin, b_lin, W1, b1, W2, b2, W3, b3, W4, b4):
    raise NotImplementedError("write your pallas kernel here")


# speedup vs baseline: 6.8529x; 6.8529x over previous
import functools

import jax
import jax.numpy as jnp
from jax import lax
from jax.experimental import pallas as pl
from jax.experimental.pallas import tpu as pltpu
from jax.experimental.pallas import tpu_sc as plsc

B = 16384
D_DENSE = 13
F = 26
V = 100000
E = 16
TOTAL = B * F

NC = 2
NS = 16
NW = NC * NS
PER_W = TOTAL // NW
CHUNK = 128
CPW = PER_W // CHUNK
GC = 13
NG = CPW // GC
GROUP_ROWS = GC * CHUNK


def _sc_gather_body(cats_hbm, tables_hbm, out_hbm, cats_v, idx_v, rows_v, sem):
    wid = lax.axis_index("s") * NC + lax.axis_index("c")
    chunk_base = wid * CPW
    row_base = wid * PER_W

    pltpu.sync_copy(cats_hbm.at[pl.ds(chunk_base, CPW)], cats_v)

    lane = lax.iota(jnp.int32, 16)

    def idx_body(j, carry):
        for l in range(CHUNK // 16):
            pos = row_base + j * CHUNK + l * 16 + lane
            field = lax.rem(pos, F)
            c = cats_v[j, pl.ds(l * 16, 16)]
            idx_v[j, pl.ds(l * 16, 16)] = c + field * V
        return carry

    lax.fori_loop(0, CPW, idx_body, 0)

    def group_body(g, carry):
        handles = []
        for j in range(GC):
            h = pltpu.async_copy(
                tables_hbm.at[idx_v.at[g * GC + j]],
                rows_v.at[pl.ds(j * CHUNK, CHUNK)],
                sem,
            )
            handles.append(h)
        for h in handles:
            h.wait()
        pltpu.sync_copy(rows_v, out_hbm.at[pl.ds(row_base + g * GROUP_ROWS, GROUP_ROWS)])
        return carry

    lax.fori_loop(0, NG, group_body, 0)


@functools.cache
def _sc_gather():
    return pl.kernel(
        _sc_gather_body,
        out_type=jax.ShapeDtypeStruct((TOTAL, E), jnp.float32),
        mesh=plsc.VectorSubcoreMesh(
            core_axis_name="c", subcore_axis_name="s",
            num_cores=NC, num_subcores=NS),
        scratch_types=[
            pltpu.VMEM((CPW, CHUNK), jnp.int32),
            pltpu.VMEM((CPW, CHUNK), jnp.int32),
            pltpu.VMEM((GROUP_ROWS, E), jnp.float32),
            pltpu.SemaphoreType.DMA,
        ],
        compiler_params=pltpu.CompilerParams(use_tc_tiling_on_sc=False),
    )


BB = 1024


def _mlp_body(xd_ref, xe_ref, w1d_ref, w1e_ref, b1_ref, w2_ref, b2_ref,
              w3_ref, b3_ref, w4_ref, b4_ref, wld_ref, wle_ref, bl_ref,
              out_ref):
    f32 = jnp.float32
    hi = jax.lax.Precision.HIGHEST
    xd = xd_ref[...]
    xe = xe_ref[...]
    h = (jnp.dot(xd, w1d_ref[...], precision=hi, preferred_element_type=f32)
         + jnp.dot(xe, w1e_ref[...], precision=hi, preferred_element_type=f32)
         + b1_ref[...])
    h = jnp.maximum(h, 0.0)
    h = jnp.maximum(jnp.dot(h, w2_ref[...], precision=hi, preferred_element_type=f32) + b2_ref[...], 0.0)
    h = jnp.maximum(jnp.dot(h, w3_ref[...], precision=hi, preferred_element_type=f32) + b3_ref[...], 0.0)
    y_deep = jnp.dot(h, w4_ref[...], precision=hi, preferred_element_type=f32) + b4_ref[...]
    y_lin = (jnp.dot(xd, wld_ref[...], precision=hi, preferred_element_type=f32)
             + jnp.dot(xe, wle_ref[...], precision=hi, preferred_element_type=f32)
             + bl_ref[...])
    out_ref[...] = jax.nn.sigmoid(y_lin + y_deep)


def _full(shape):
    return pl.BlockSpec(shape, lambda i: (0, 0))


def kernel(dense, cats, tables, W_lin, b_lin, W1, b1, W2, b2, W3, b3, W4, b4):
    cats2d = cats.reshape(TOTAL // CHUNK, CHUNK)
    tables2d = tables.reshape(F * V, E)
    emb = _sc_gather()(cats2d, tables2d)
    xe = emb.reshape(B, F * E)

    w1d, w1e = W1[:D_DENSE], W1[D_DENSE:]
    wld, wle = W_lin[:D_DENSE], W_lin[D_DENSE:]

    mlp = pl.pallas_call(
        _mlp_body,
        grid=(B // BB,),
        in_specs=[
            pl.BlockSpec((BB, D_DENSE), lambda i: (i, 0)),
            pl.BlockSpec((BB, F * E), lambda i: (i, 0)),
            _full((D_DENSE, 256)), _full((F * E, 256)), _full((1, 256)),
            _full((256, 128)), _full((1, 128)),
            _full((128, 64)), _full((1, 64)),
            _full((64, 1)), _full((1, 1)),
            _full((D_DENSE, 1)), _full((F * E, 1)), _full((1, 1)),
        ],
        out_specs=pl.BlockSpec((BB, 1), lambda i: (i, 0)),
        out_shape=jax.ShapeDtypeStruct((B, 1), jnp.float32),
    )
    return mlp(dense, xe,
               w1d, w1e, b1.reshape(1, -1),
               W2, b2.reshape(1, -1),
               W3, b3.reshape(1, -1),
               W4, b4.reshape(1, -1),
               wld, wle, b_lin.reshape(1, -1))
